# baseline (speedup 1.0000x reference)
import jax
import jax.numpy as jnp
from jax import lax
from jax.experimental import pallas as pl
from jax.experimental.pallas import tpu as pltpu
from jax.experimental.pallas import tpu_sc as plsc

N_SHIFT = 2

B, C, T = 64, 128, 4096
SCB = 16
NC, NS = 2, 16
NW = NC * NS
R = 8
CHPB = C // R
K = SCB * CHPB // NW
L = 16
LW = T + 2 * L


def _shift_constants():
    return (
        2, 2, -1, 2, 2, 2, 0, 0, 2, -1, 0, 2, -2, -1, 0, -2,
        1, 2, -2, 2, 0, 1, 1, 0, 2, -1, 0, -1, 0, 2, 2, 0,
        0, 1, -1, 2, 0, 2, 1, 1, 2, -1, -2, 2, -2, 0, -1, 2,
        0, 1, 1, -2, 0, 1, 2, 2, -1, -2, 0, -1, -2, -2, 2, -2,
    )


def _sc_body(x_hbm, out_hbm, vin0, vin1, lin, isem0, isem1, osem0, osem1):
    cid = lax.axis_index("c")
    sid = lax.axis_index("s")
    wid = sid * NC + cid

    shifts = _shift_constants()

    def dsel(b):
        dd = jnp.int32(L - shifts[0])
        for w in range(1, SCB):
            dd = jnp.where(b == w, jnp.int32(L - shifts[w]), dd)
        return dd

    vin = (vin0, vin1)
    isem = (isem0, isem1)
    osem = (osem0, osem1)

    def slices(kk):
        g = wid * K + kk
        b = g // CHPB
        r0 = (g % CHPB) * R
        return b, r0

    def start_in(kk, p):
        b, r0 = slices(kk)
        pltpu.make_async_copy(x_hbm.at[b, pl.ds(r0, R), :], vin[p], isem[p]).start()

    def wait_in(kk, p):
        b, r0 = slices(kk)
        pltpu.make_async_copy(x_hbm.at[b, pl.ds(r0, R), :], vin[p], isem[p]).wait()

    def start_out(kk, p):
        b, r0 = slices(kk)
        pltpu.make_async_copy(vin[p], out_hbm.at[b, pl.ds(r0, R), :], osem[p]).start()

    def wait_out(kk, p):
        b, r0 = slices(kk)
        pltpu.make_async_copy(vin[p], out_hbm.at[b, pl.ds(r0, R), :], osem[p]).wait()

    def pass1(p):
        for r in range(R):
            base = r * LW

            @plsc.parallel_loop(0, T // L, unroll=8)
            def detile(i, r=r, p=p, base=base):
                lin[pl.ds(base + L + i * L, L)] = vin[p][r, pl.ds(i * L, L)]

            lin[pl.ds(base, L)] = vin[p][r, pl.ds(T - L, L)]
            lin[pl.ds(base + L + T, L)] = vin[p][r, pl.ds(0, L)]

    def pass2(p, d):
        for r in range(R):
            base = r * LW

            @plsc.parallel_loop(0, T // L, unroll=8)
            def shuf(i, r=r, p=p, base=base, d=d):
                vin[p][r, pl.ds(i * L, L)] = lin[pl.ds(base + i * L + d, L)]

    start_in(0, 0)
    start_in(1, 1)

    def step(kk, _):
        p = lax.rem(kk, 2)

        def proc(p):
            b, _r0 = slices(kk)
            d = dsel(b)
            wait_in(kk, p)
            pass1(p)
            @pl.when(jnp.logical_and(kk >= 1, kk + 1 < K))
            def _():
                wait_out(kk - 1, 1 - p)
                start_in(kk + 1, 1 - p)

            pass2(p, d)
            start_out(kk, p)

        @pl.when(p == 0)
        def _():
            proc(0)

        @pl.when(p == 1)
        def _():
            proc(1)

        return _

    lax.fori_loop(0, K, step, None)

    wait_out(K - 2, (K - 2) % 2)
    wait_out(K - 1, (K - 1) % 2)


def _sc_kernel(input):
    mesh = plsc.VectorSubcoreMesh(
        core_axis_name="c", subcore_axis_name="s", num_cores=NC, num_subcores=NS
    )
    f = pl.kernel(
        _sc_body,
        out_type=jax.ShapeDtypeStruct((SCB, C, T), jnp.float32),
        mesh=mesh,
        scratch_types=[
            pltpu.VMEM((R, T), jnp.float32),
            pltpu.VMEM((R, T), jnp.float32),
            pltpu.VMEM((R * LW,), jnp.float32),
            pltpu.SemaphoreType.DMA,
            pltpu.SemaphoreType.DMA,
            pltpu.SemaphoreType.DMA,
            pltpu.SemaphoreType.DMA,
        ],
    )
    return f(input)


TCSPLIT = 4


def _tc_roll_body(x_ref, o_ref):
    shifts = _shift_constants()
    b = pl.program_id(0) // TCSPLIT + SCB
    s = jnp.int32(shifts[SCB])
    for w in range(SCB + 1, B):
        s = jnp.where(b == w, jnp.int32(shifts[w]), s)
    o_ref[0] = pltpu.roll(x_ref[0], s, axis=1)


def _tc_kernel(input):
    cs = C // TCSPLIT
    return pl.pallas_call(
        _tc_roll_body,
        grid=((B - SCB) * TCSPLIT,),
        in_specs=[
            pl.BlockSpec((1, cs, T), lambda i: (i // TCSPLIT + SCB, i % TCSPLIT, 0))
        ],
        out_specs=pl.BlockSpec(
            (1, cs, T), lambda i: (i // TCSPLIT + SCB, i % TCSPLIT, 0)
        ),
        out_shape=jax.ShapeDtypeStruct((B, C, T), jnp.float32),
    )(input)


@jax.jit
def kernel(input):
    sc_out = _sc_kernel(input)
    tc_out = _tc_kernel(input)
    return lax.dynamic_update_slice(tc_out, sc_out, (0, 0, 0))


# speedup vs baseline: 1.4807x; 1.4807x over previous
import jax
import jax.numpy as jnp
from jax import lax
from jax.experimental import pallas as pl
from jax.experimental.pallas import tpu as pltpu
from jax.experimental.pallas import tpu_sc as plsc

N_SHIFT = 2

B, C, T = 64, 128, 4096
SCB = 16
NC, NS = 2, 16
NW = NC * NS
R = 8
CHPB = C // R
K = SCB * CHPB // NW
L = 16
LW = T + 2 * L


def _shift_constants():
    return (
        2, 2, -1, 2, 2, 2, 0, 0, 2, -1, 0, 2, -2, -1, 0, -2,
        1, 2, -2, 2, 0, 1, 1, 0, 2, -1, 0, -1, 0, 2, 2, 0,
        0, 1, -1, 2, 0, 2, 1, 1, 2, -1, -2, 2, -2, 0, -1, 2,
        0, 1, 1, -2, 0, 1, 2, 2, -1, -2, 0, -1, -2, -2, 2, -2,
    )


def _sc_body(x_hbm, out_hbm, vin0, vin1, lin, isem0, isem1, osem0, osem1):
    cid = lax.axis_index("c")
    sid = lax.axis_index("s")
    wid = sid * NC + cid

    shifts = _shift_constants()

    def dsel(b):
        dd = jnp.int32(L - shifts[0])
        for w in range(1, SCB):
            dd = jnp.where(b == w, jnp.int32(L - shifts[w]), dd)
        return dd

    vin = (vin0, vin1)
    isem = (isem0, isem1)
    osem = (osem0, osem1)

    def slices(kk):
        g = wid * K + kk
        b = g // CHPB
        r0 = (g % CHPB) * R
        return b, r0

    def start_in(kk, p):
        b, r0 = slices(kk)
        pltpu.make_async_copy(x_hbm.at[b, pl.ds(r0, R), :], vin[p], isem[p]).start()

    def wait_in(kk, p):
        b, r0 = slices(kk)
        pltpu.make_async_copy(x_hbm.at[b, pl.ds(r0, R), :], vin[p], isem[p]).wait()

    def start_out(kk, p):
        b, r0 = slices(kk)
        pltpu.make_async_copy(vin[p], out_hbm.at[b, pl.ds(r0, R), :], osem[p]).start()

    def wait_out(kk, p):
        b, r0 = slices(kk)
        pltpu.make_async_copy(vin[p], out_hbm.at[b, pl.ds(r0, R), :], osem[p]).wait()

    def pass1(p):
        for r in range(R):
            base = r * LW

            @plsc.parallel_loop(0, T // L, unroll=8)
            def detile(i, r=r, p=p, base=base):
                lin[pl.ds(base + L + i * L, L)] = vin[p][r, pl.ds(i * L, L)]

            lin[pl.ds(base, L)] = vin[p][r, pl.ds(T - L, L)]
            lin[pl.ds(base + L + T, L)] = vin[p][r, pl.ds(0, L)]

    def pass2(p, d):
        for r in range(R):
            base = r * LW

            @plsc.parallel_loop(0, T // L, unroll=8)
            def shuf(i, r=r, p=p, base=base, d=d):
                vin[p][r, pl.ds(i * L, L)] = lin[pl.ds(base + i * L + d, L)]

    start_in(0, 0)
    start_in(1, 1)

    def step(kk, _):
        p = lax.rem(kk, 2)

        def proc(p):
            b, _r0 = slices(kk)
            d = dsel(b)
            wait_in(kk, p)
            pass1(p)
            @pl.when(jnp.logical_and(kk >= 1, kk + 1 < K))
            def _():
                wait_out(kk - 1, 1 - p)
                start_in(kk + 1, 1 - p)

            pass2(p, d)
            start_out(kk, p)

        @pl.when(p == 0)
        def _():
            proc(0)

        @pl.when(p == 1)
        def _():
            proc(1)

        return _

    lax.fori_loop(0, K, step, None)

    wait_out(K - 2, (K - 2) % 2)
    wait_out(K - 1, (K - 1) % 2)


def _sc_kernel(input):
    mesh = plsc.VectorSubcoreMesh(
        core_axis_name="c", subcore_axis_name="s", num_cores=NC, num_subcores=NS
    )
    f = pl.kernel(
        _sc_body,
        out_type=jax.ShapeDtypeStruct((SCB, C, T), jnp.float32),
        mesh=mesh,
        scratch_types=[
            pltpu.VMEM((R, T), jnp.float32),
            pltpu.VMEM((R, T), jnp.float32),
            pltpu.VMEM((R * LW,), jnp.float32),
            pltpu.SemaphoreType.DMA,
            pltpu.SemaphoreType.DMA,
            pltpu.SemaphoreType.DMA,
            pltpu.SemaphoreType.DMA,
        ],
    )
    return f(input)


TCSPLIT = 1


def _tc_roll_body(x_ref, o_ref):
    shifts = _shift_constants()
    b = pl.program_id(0) // TCSPLIT + SCB
    s = jnp.int32(shifts[SCB])
    for w in range(SCB + 1, B):
        s = jnp.where(b == w, jnp.int32(shifts[w]), s)
    o_ref[0] = pltpu.roll(x_ref[0], s, axis=1)


def _tc_kernel(input):
    cs = C // TCSPLIT
    return pl.pallas_call(
        _tc_roll_body,
        grid=((B - SCB) * TCSPLIT,),
        in_specs=[
            pl.BlockSpec((1, cs, T), lambda i: (i // TCSPLIT + SCB, i % TCSPLIT, 0))
        ],
        out_specs=pl.BlockSpec(
            (1, cs, T), lambda i: (i // TCSPLIT + SCB, i % TCSPLIT, 0)
        ),
        out_shape=jax.ShapeDtypeStruct((B, C, T), jnp.float32),
    )(input)


@jax.jit
def kernel(input):
    sc_out = _sc_kernel(input)
    tc_out = _tc_kernel(input)
    return lax.dynamic_update_slice(tc_out, sc_out, (0, 0, 0))
